# baseline (speedup 1.0000x reference)
import functools

import jax
import jax.numpy as jnp
from jax import lax
from jax.experimental import pallas as pl
from jax.experimental.pallas import tpu as pltpu
from jax.experimental.pallas import tpu_sc as plsc

N = 10000
E = 320000
D = 128
G = 64

L = 16
NC = 2
NS = 16
NW = NC * NS

NP = 10240
CH = 64
GS = 32
CPW = 160
TOTC = 5120
CS = 96
CF = 224
SLOW_CID = 1
EPW = CPW * CH
EPAD = NW * EPW
RPT = NP // NS
MCH = 128

SEG = 80
AW = L * SEG
ACCP = 3 * AW
OUTP = 80
CBN = 2 * D + L

_mesh2 = plsc.VectorSubcoreMesh(
    core_axis_name="c", subcore_axis_name="s", num_cores=NC, num_subcores=NS)
_mesh1 = plsc.VectorSubcoreMesh(
    core_axis_name="c", subcore_axis_name="s", num_cores=1, num_subcores=NS)


def _deg_body(dsts, deg_out, didx, ones_b, zb, sem, shared):
  cid = lax.axis_index("c")
  sid = lax.axis_index("s")
  w = sid * NC + cid
  zeros = jnp.zeros((L,), jnp.float32)
  ones = jnp.ones((L,), jnp.float32)
  for i in range(RPT // L):
    zb[pl.ds(i * L, L)] = zeros
  for i in range(CH // L):
    ones_b[pl.ds(i * L, L)] = ones
  pltpu.sync_copy(zb, shared.at[pl.ds(sid * RPT, RPT)])
  pltpu.sync_copy(dsts.at[w], didx)
  plsc.subcore_barrier()

  def step(j, c):
    pltpu.sync_copy(ones_b, shared.at[didx.at[j]], add=True)
    return c
  lax.fori_loop(0, CPW, step, 0)
  plsc.subcore_barrier()

  @pl.when(sid == 0)
  def _():
    pltpu.async_copy(shared, deg_out.at[pl.ds(cid * NP, NP)], sem).wait()


def _deg_call(dsts3):
  f = pl.kernel(
      _deg_body,
      out_type=jax.ShapeDtypeStruct((NC * NP,), jnp.float32),
      mesh=_mesh2,
      scratch_types=[
          pltpu.VMEM((CPW, CH), jnp.int32),
          pltpu.VMEM((CH,), jnp.float32),
          pltpu.VMEM((RPT,), jnp.float32),
          pltpu.SemaphoreType.DMA,
          pltpu.VMEM_SHARED((NP,), jnp.float32),
      ],
  )
  return f(dsts3)


def _tc1_body(degT, x, bp, xs_out, dinv_out, wd_out):
  d = degT[:, 0:1] + degT[:, 1:2] + 1.0
  dinv = lax.rsqrt(d)
  dinv_out[...] = dinv
  bits = lax.bitcast_convert_type(dinv, jnp.int32)
  wd_out[...] = lax.bitcast_convert_type(
      lax.bitwise_or(lax.bitwise_and(bits, -128), bp[...]), jnp.float32)
  xs_out[...] = x[...] * dinv


def _tc1_call(degT, xpad, batchp2):
  blk = NP // 10
  return pl.pallas_call(
      _tc1_body,
      grid=(10,),
      in_specs=[
          pl.BlockSpec((blk, 2), lambda i: (i, 0)),
          pl.BlockSpec((blk, D), lambda i: (i, 0)),
          pl.BlockSpec((blk, 1), lambda i: (i, 0)),
      ],
      out_specs=[
          pl.BlockSpec((blk, D), lambda i: (i, 0)),
          pl.BlockSpec((blk, 1), lambda i: (i, 0)),
          pl.BlockSpec((blk, 1), lambda i: (i, 0)),
      ],
      out_shape=[
          jax.ShapeDtypeStruct((NP, D), jnp.float32),
          jax.ShapeDtypeStruct((NP, 1), jnp.float32),
          jax.ShapeDtypeStruct((NP, 1), jnp.float32),
      ],
  )(degT, xpad, batchp2)


_NBUF = 4


def _gat_body(xs_h, srcs, dsts, s_out, sidx, didx, r0, r1, r2, r3, zb, sem,
              gsem, ssem, shared):
  cid = lax.axis_index("c")
  sid = lax.axis_index("s")
  ngrp = jnp.where(cid == SLOW_CID, CS // GS, CF // GS)
  coff = jnp.where(cid == SLOW_CID, sid * CS, NS * CS + sid * CF)
  zeros = jnp.zeros((L,), jnp.float32)
  for r in range(16):
    for c2 in range(D // L):
      zb[r, pl.ds(c2 * L, L)] = zeros

  def zstep(t, c):
    pltpu.sync_copy(zb, shared.at[pl.ds(sid * RPT + t * 16, 16)])
    return c
  lax.fori_loop(0, RPT // 16, zstep, 0)
  plsc.subcore_barrier()

  rows = [r0, r1, r2, r3]

  def _gth(j, b):
    return pltpu.make_async_copy(xs_h.at[sidx.at[j]], rows[b], gsem)

  def _sct(j, b):
    return pltpu.make_async_copy(rows[b], shared.at[didx.at[j]], ssem)

  def ggroup(g, c):
    pltpu.sync_copy(srcs.at[pl.ds(coff + g * GS, GS)], sidx)
    pltpu.sync_copy(dsts.at[pl.ds(coff + g * GS, GS)], didx)
    for b in range(_NBUF):
      _gth(b, b).start()

    def sub(sg, c2):
      base = sg * _NBUF
      for b in range(_NBUF):
        _gth(base + b, b).wait()
        _sct(base + b, b).start(add=True)
      for b in range(_NBUF):
        _sct(base + b, b).wait()
        _gth(base + _NBUF + b, b).start()
      return c2
    lax.fori_loop(0, GS // _NBUF - 1, sub, 0)
    base = GS - _NBUF
    for b in range(_NBUF):
      _gth(base + b, b).wait()
      _sct(base + b, b).start(add=True)
    for b in range(_NBUF):
      _sct(base + b, b).wait()
    return c
  lax.fori_loop(0, ngrp, ggroup, 0)
  plsc.subcore_barrier()
  pltpu.async_copy(
      shared.at[pl.ds(sid * RPT, RPT)],
      s_out.at[pl.ds(cid * NP + sid * RPT, RPT)], sem).wait()


def _gat_call(xs, srcs3, dsts3):
  f = pl.kernel(
      _gat_body,
      out_type=jax.ShapeDtypeStruct((NC * NP, D), jnp.float32),
      mesh=_mesh2,
      scratch_types=[
          pltpu.VMEM((GS, CH), jnp.int32),
          pltpu.VMEM((GS, CH), jnp.int32),
          pltpu.VMEM((CH, D), jnp.float32),
          pltpu.VMEM((CH, D), jnp.float32),
          pltpu.VMEM((CH, D), jnp.float32),
          pltpu.VMEM((CH, D), jnp.float32),
          pltpu.VMEM((16, D), jnp.float32),
          pltpu.SemaphoreType.DMA,
          pltpu.SemaphoreType.DMA,
          pltpu.SemaphoreType.DMA,
          pltpu.VMEM_SHARED((NP, D), jnp.float32),
      ],
  )
  return f(xs, srcs3, dsts3)


def _tc2_body(s0, s1, xs, dinvc, w1, b1r, w2, wfc, out):
  hi = jax.lax.Precision.HIGHEST
  u = jnp.dot(w2[...], wfc[...], precision=hi,
              preferred_element_type=jnp.float32)
  a = (s0[...] + s1[...] + xs[...]) * dinvc[...]
  h1 = jnp.dot(a, w1[...], precision=hi,
               preferred_element_type=jnp.float32) + b1r[...]
  h1 = jnp.maximum(h1, 0.0)
  v = jnp.dot(h1, u, precision=hi, preferred_element_type=jnp.float32)
  out[...] = v * dinvc[...]


def _tc2_call(s_flat, xs, dinv, W1, b1r, W2, Wfc):
  blk = NP // 10
  nb = NP // blk
  return pl.pallas_call(
      _tc2_body,
      grid=(10,),
      in_specs=[
          pl.BlockSpec((blk, D), lambda i: (i, 0)),
          pl.BlockSpec((blk, D), lambda i, _nb=nb: (i + _nb, 0)),
          pl.BlockSpec((blk, D), lambda i: (i, 0)),
          pl.BlockSpec((blk, 1), lambda i: (i, 0)),
          pl.BlockSpec((D, D), lambda i: (0, 0)),
          pl.BlockSpec((1, D), lambda i: (0, 0)),
          pl.BlockSpec((D, D), lambda i: (0, 0)),
          pl.BlockSpec((D, 1), lambda i: (0, 0)),
      ],
      out_specs=pl.BlockSpec((blk, 1), lambda i: (i, 0)),
      out_shape=jax.ShapeDtypeStruct((NP, 1), jnp.float32),
  )(s_flat, s_flat, xs, dinv, W1, b1r, W2, Wfc)


def _fin_body(vs_h, wd_h, srcs_f, dsts_f, cb_h, m_out, vs_l, wd_l, sidx,
              didx, acc, ident, fin, cb_l, outm, sem, shared):
  sid = lax.axis_index("s")
  zeros = jnp.zeros((L,), jnp.float32)
  ones = jnp.ones((L,), jnp.float32)
  iota = lax.iota(jnp.int32, L)
  m127 = jnp.full((L,), 127, jnp.int32)

  def zstep(i, c):
    acc[pl.ds(i * L, L)] = zeros
    return c
  lax.fori_loop(0, ACCP // L, zstep, 0)
  stripe = ACCP // NS
  pltpu.sync_copy(acc.at[pl.ds(0, stripe)],
                  shared.at[pl.ds(sid * stripe, stripe)])
  for t in range(ACCP // MCH):
    for kk in range(MCH // L):
      ident[t, pl.ds(kk * L, L)] = t * MCH + kk * L + iota
  pltpu.sync_copy(vs_h, vs_l)
  pltpu.sync_copy(wd_h, wd_l)
  pltpu.sync_copy(cb_h, cb_l)
  plsc.subcore_barrier()

  for half in range(2):
    w = sid * 2 + half
    pltpu.sync_copy(srcs_f.at[pl.ds(w * EPW, EPW)], sidx)
    pltpu.sync_copy(dsts_f.at[pl.ds(w * EPW, EPW)], didx)

    def estep(j, c):
      for kk in range(CH // L):
        off = j * CH + kk * L
        sv = sidx[pl.ds(off, L)]
        dv = didx[pl.ds(off, L)]
        vsrc = plsc.load_gather(vs_l, [sv])
        wdv = plsc.load_gather(wd_l, [dv])
        bv = lax.bitwise_and(plsc.bitcast(wdv, jnp.int32), m127)
        idx = iota * SEG + bv
        plsc.addupdate_scatter(acc, [idx], vsrc * wdv)
      return c
    lax.fori_loop(0, CPW, estep, 0)

  npt = NP // NS

  def nstep(i, c):
    base = sid * npt + i * L
    wdv = wd_l[pl.ds(base, L)]
    vv = vs_l[pl.ds(base, L)]
    bv = lax.bitwise_and(plsc.bitcast(wdv, jnp.int32), m127)
    idx = iota * SEG + bv
    plsc.addupdate_scatter(acc, [idx + AW], wdv * vv)
    plsc.addupdate_scatter(acc, [idx + 2 * AW], ones)
    return c
  lax.fori_loop(0, npt // L, nstep, 0)

  def mstep(t, c):
    pltpu.sync_copy(acc.at[pl.ds(t * MCH, MCH)], shared.at[ident.at[t]],
                    add=True)
    return c
  lax.fori_loop(0, ACCP // MCH, mstep, 0)
  plsc.subcore_barrier()

  @pl.when(sid == 0)
  def _():
    pltpu.sync_copy(shared, fin)
    dacc = zeros
    for c in range(D // L):
      dacc = dacc + cb_l[pl.ds(c * L, L)] * cb_l[pl.ds(D + c * L, L)]
    cbw = jnp.sum(dacc)
    bfcs = cb_l[pl.ds(2 * D, L)][0]
    for i in range(OUTP // L):
      pe = zeros
      ps = zeros
      cnt = zeros
      for s in range(L):
        pe = pe + fin[pl.ds(s * SEG + i * L, L)]
        ps = ps + fin[pl.ds(AW + s * SEG + i * L, L)]
        cnt = cnt + fin[pl.ds(2 * AW + s * SEG + i * L, L)]
      m = (pe + ps) / jnp.maximum(cnt, 1.0)
      outm[pl.ds(i * L, L)] = m + jnp.where(cnt > 0, cbw, 0.0) + bfcs
    pltpu.async_copy(outm, m_out, sem).wait()


def _fin_call(vs, wd, srcs_f, dsts_f, cbv):
  f = pl.kernel(
      _fin_body,
      out_type=jax.ShapeDtypeStruct((OUTP,), jnp.float32),
      mesh=_mesh1,
      compiler_params=pltpu.CompilerParams(needs_layout_passes=False),
      scratch_types=[
          pltpu.VMEM((NP,), jnp.float32),
          pltpu.VMEM((NP,), jnp.float32),
          pltpu.VMEM((EPW,), jnp.int32),
          pltpu.VMEM((EPW,), jnp.int32),
          pltpu.VMEM((ACCP,), jnp.float32),
          pltpu.VMEM((ACCP // MCH, MCH), jnp.int32),
          pltpu.VMEM((ACCP,), jnp.float32),
          pltpu.VMEM((CBN,), jnp.float32),
          pltpu.VMEM((OUTP,), jnp.float32),
          pltpu.SemaphoreType.DMA,
          pltpu.VMEM_SHARED((ACCP,), jnp.float32),
      ],
  )
  return f(vs, wd, srcs_f, dsts_f, cbv)


@jax.jit
def kernel(x, edge_index, batch, W1, b1, W2, b2, Wfc, bfc):
  src = edge_index[0]
  dst = edge_index[1]
  pad = jnp.full((EPAD - E,), N, dtype=jnp.int32)
  src_f = jnp.concatenate([src, pad])
  dst_f = jnp.concatenate([dst, pad])
  srcs3 = src_f.reshape(NW, CPW, CH)
  dsts3 = dst_f.reshape(NW, CPW, CH)
  xpad = jnp.pad(x, ((0, NP - N), (0, 0)))
  batchp = jnp.pad(batch, (0, NP - N), constant_values=G)
  cbv = jnp.concatenate(
      [b2, Wfc.reshape(D), jnp.broadcast_to(bfc, (CBN - 2 * D,))])

  deg_flat = _deg_call(dsts3)
  degT = deg_flat.reshape(NC, NP).T
  xs, dinv, wd = _tc1_call(degT, xpad, batchp.reshape(NP, 1))
  s_flat = _gat_call(xs, src_f.reshape(TOTC, CH), dst_f.reshape(TOTC, CH))
  vs2 = _tc2_call(s_flat, xs, dinv, W1, b1.reshape(1, D), W2, Wfc)
  m = _fin_call(vs2.reshape(NP), wd.reshape(NP), src_f, dst_f, cbv)
  return m[:G][:, None]


# speedup vs baseline: 1.0844x; 1.0844x over previous
import functools

import jax
import jax.numpy as jnp
from jax import lax
from jax.experimental import pallas as pl
from jax.experimental.pallas import tpu as pltpu
from jax.experimental.pallas import tpu_sc as plsc

N = 10000
E = 320000
D = 128
G = 64

L = 16
NC = 2
NS = 16
NW = NC * NS

NP = 10240
CH = 64
GS = 32
CPW = 160
TOTC = 5120
CS = 160
CF = 160
SLOW_CID = 0
EPW = CPW * CH
EPAD = NW * EPW
RPT = NP // NS
MCH = 128

SEG = 80
AW = L * SEG
ACCP = 3 * AW
OUTP = 80
CBN = 2 * D + L

_mesh2 = plsc.VectorSubcoreMesh(
    core_axis_name="c", subcore_axis_name="s", num_cores=NC, num_subcores=NS)
_mesh1 = plsc.VectorSubcoreMesh(
    core_axis_name="c", subcore_axis_name="s", num_cores=1, num_subcores=NS)


def _deg_body(dsts, deg_out, didx, ones_b, zb, sem, shared):
  cid = lax.axis_index("c")
  sid = lax.axis_index("s")
  w = sid * NC + cid
  zeros = jnp.zeros((L,), jnp.float32)
  ones = jnp.ones((L,), jnp.float32)
  for i in range(RPT // L):
    zb[pl.ds(i * L, L)] = zeros
  for i in range(CH // L):
    ones_b[pl.ds(i * L, L)] = ones
  pltpu.sync_copy(zb, shared.at[pl.ds(sid * RPT, RPT)])
  pltpu.sync_copy(dsts.at[w], didx)
  plsc.subcore_barrier()

  def step(j, c):
    pltpu.sync_copy(ones_b, shared.at[didx.at[j]], add=True)
    return c
  lax.fori_loop(0, CPW, step, 0)
  plsc.subcore_barrier()

  @pl.when(sid == 0)
  def _():
    pltpu.async_copy(shared, deg_out.at[pl.ds(cid * NP, NP)], sem).wait()


def _deg_call(dsts3):
  f = pl.kernel(
      _deg_body,
      out_type=jax.ShapeDtypeStruct((NC * NP,), jnp.float32),
      mesh=_mesh2,
      scratch_types=[
          pltpu.VMEM((CPW, CH), jnp.int32),
          pltpu.VMEM((CH,), jnp.float32),
          pltpu.VMEM((RPT,), jnp.float32),
          pltpu.SemaphoreType.DMA,
          pltpu.VMEM_SHARED((NP,), jnp.float32),
      ],
  )
  return f(dsts3)


def _tc1_body(degT, x, bp, xs_out, dinv_out, wd_out):
  d = degT[:, 0:1] + degT[:, 1:2] + 1.0
  dinv = lax.rsqrt(d)
  dinv_out[...] = dinv
  bits = lax.bitcast_convert_type(dinv, jnp.int32)
  wd_out[...] = lax.bitcast_convert_type(
      lax.bitwise_or(lax.bitwise_and(bits, -128), bp[...]), jnp.float32)
  xs_out[...] = x[...] * dinv


def _tc1_call(degT, xpad, batchp2):
  blk = NP // 10
  return pl.pallas_call(
      _tc1_body,
      grid=(10,),
      in_specs=[
          pl.BlockSpec((blk, 2), lambda i: (i, 0)),
          pl.BlockSpec((blk, D), lambda i: (i, 0)),
          pl.BlockSpec((blk, 1), lambda i: (i, 0)),
      ],
      out_specs=[
          pl.BlockSpec((blk, D), lambda i: (i, 0)),
          pl.BlockSpec((blk, 1), lambda i: (i, 0)),
          pl.BlockSpec((blk, 1), lambda i: (i, 0)),
      ],
      out_shape=[
          jax.ShapeDtypeStruct((NP, D), jnp.float32),
          jax.ShapeDtypeStruct((NP, 1), jnp.float32),
          jax.ShapeDtypeStruct((NP, 1), jnp.float32),
      ],
  )(degT, xpad, batchp2)


_NBUF = 4


def _gat_body(xs_h, srcs, dsts, s_out, sidx, didx, r0, r1, r2, r3, zb, sem,
              gsem, ssem, shared):
  cid = lax.axis_index("c")
  sid = lax.axis_index("s")
  ngrp = jnp.where(cid == SLOW_CID, CS // GS, CF // GS)
  coff = jnp.where(cid == SLOW_CID, sid * CS, NS * CS + sid * CF)
  zeros = jnp.zeros((L,), jnp.float32)
  for r in range(16):
    for c2 in range(D // L):
      zb[r, pl.ds(c2 * L, L)] = zeros

  def zstep(t, c):
    pltpu.sync_copy(zb, shared.at[pl.ds(sid * RPT + t * 16, 16)])
    return c
  lax.fori_loop(0, RPT // 16, zstep, 0)
  plsc.subcore_barrier()

  rows = [r0, r1, r2, r3]

  def _gth(j, b):
    return pltpu.make_async_copy(xs_h.at[sidx.at[j]], rows[b], gsem)

  def _sct(j, b):
    return pltpu.make_async_copy(rows[b], shared.at[didx.at[j]], ssem)

  def ggroup(g, c):
    pltpu.sync_copy(srcs.at[pl.ds(coff + g * GS, GS)], sidx)
    pltpu.sync_copy(dsts.at[pl.ds(coff + g * GS, GS)], didx)
    for b in range(_NBUF):
      _gth(b, b).start()

    def sub(sg, c2):
      base = sg * _NBUF
      for b in range(_NBUF):
        _gth(base + b, b).wait()
        _sct(base + b, b).start(add=True)
      for b in range(_NBUF):
        _sct(base + b, b).wait()
        _gth(base + _NBUF + b, b).start()
      return c2
    lax.fori_loop(0, GS // _NBUF - 1, sub, 0)
    base = GS - _NBUF
    for b in range(_NBUF):
      _gth(base + b, b).wait()
      _sct(base + b, b).start(add=True)
    for b in range(_NBUF):
      _sct(base + b, b).wait()
    return c
  lax.fori_loop(0, ngrp, ggroup, 0)
  plsc.subcore_barrier()
  pltpu.async_copy(
      shared.at[pl.ds(sid * RPT, RPT)],
      s_out.at[pl.ds(cid * NP + sid * RPT, RPT)], sem).wait()


def _gat_call(xs, srcs3, dsts3):
  f = pl.kernel(
      _gat_body,
      out_type=jax.ShapeDtypeStruct((NC * NP, D), jnp.float32),
      mesh=_mesh2,
      scratch_types=[
          pltpu.VMEM((GS, CH), jnp.int32),
          pltpu.VMEM((GS, CH), jnp.int32),
          pltpu.VMEM((CH, D), jnp.float32),
          pltpu.VMEM((CH, D), jnp.float32),
          pltpu.VMEM((CH, D), jnp.float32),
          pltpu.VMEM((CH, D), jnp.float32),
          pltpu.VMEM((16, D), jnp.float32),
          pltpu.SemaphoreType.DMA,
          pltpu.SemaphoreType.DMA,
          pltpu.SemaphoreType.DMA,
          pltpu.VMEM_SHARED((NP, D), jnp.float32),
      ],
  )
  return f(xs, srcs3, dsts3)


def _tc2_body(s0, s1, xs, dinvc, w1, b1r, w2, wfc, out):
  hi = jax.lax.Precision.HIGHEST
  u = jnp.dot(w2[...], wfc[...], precision=hi,
              preferred_element_type=jnp.float32)
  a = (s0[...] + s1[...] + xs[...]) * dinvc[...]
  h1 = jnp.dot(a, w1[...], precision=hi,
               preferred_element_type=jnp.float32) + b1r[...]
  h1 = jnp.maximum(h1, 0.0)
  v = jnp.dot(h1, u, precision=hi, preferred_element_type=jnp.float32)
  out[...] = v * dinvc[...]


def _tc2_call(s_flat, xs, dinv, W1, b1r, W2, Wfc):
  blk = NP // 10
  nb = NP // blk
  return pl.pallas_call(
      _tc2_body,
      grid=(10,),
      in_specs=[
          pl.BlockSpec((blk, D), lambda i: (i, 0)),
          pl.BlockSpec((blk, D), lambda i, _nb=nb: (i + _nb, 0)),
          pl.BlockSpec((blk, D), lambda i: (i, 0)),
          pl.BlockSpec((blk, 1), lambda i: (i, 0)),
          pl.BlockSpec((D, D), lambda i: (0, 0)),
          pl.BlockSpec((1, D), lambda i: (0, 0)),
          pl.BlockSpec((D, D), lambda i: (0, 0)),
          pl.BlockSpec((D, 1), lambda i: (0, 0)),
      ],
      out_specs=pl.BlockSpec((blk, 1), lambda i: (i, 0)),
      out_shape=jax.ShapeDtypeStruct((NP, 1), jnp.float32),
  )(s_flat, s_flat, xs, dinv, W1, b1r, W2, Wfc)


def _fin_body(vs_h, wd_h, srcs_f, dsts_f, cb_h, m_out, vs_l, wd_l, sidx,
              didx, acc, ident, fin, cb_l, outm, sem, shared):
  sid = lax.axis_index("s")
  zeros = jnp.zeros((L,), jnp.float32)
  ones = jnp.ones((L,), jnp.float32)
  iota = lax.iota(jnp.int32, L)
  m127 = jnp.full((L,), 127, jnp.int32)

  def zstep(i, c):
    acc[pl.ds(i * L, L)] = zeros
    return c
  lax.fori_loop(0, ACCP // L, zstep, 0)
  stripe = ACCP // NS
  pltpu.sync_copy(acc.at[pl.ds(0, stripe)],
                  shared.at[pl.ds(sid * stripe, stripe)])
  for t in range(ACCP // MCH):
    for kk in range(MCH // L):
      ident[t, pl.ds(kk * L, L)] = t * MCH + kk * L + iota
  pltpu.sync_copy(vs_h, vs_l)
  pltpu.sync_copy(wd_h, wd_l)
  pltpu.sync_copy(cb_h, cb_l)
  plsc.subcore_barrier()

  for half in range(2):
    w = sid * 2 + half
    pltpu.sync_copy(srcs_f.at[pl.ds(w * EPW, EPW)], sidx)
    pltpu.sync_copy(dsts_f.at[pl.ds(w * EPW, EPW)], didx)

    def estep(j, c):
      for kk in range(CH // L):
        off = j * CH + kk * L
        sv = sidx[pl.ds(off, L)]
        dv = didx[pl.ds(off, L)]
        vsrc = plsc.load_gather(vs_l, [sv])
        wdv = plsc.load_gather(wd_l, [dv])
        bv = lax.bitwise_and(plsc.bitcast(wdv, jnp.int32), m127)
        idx = iota * SEG + bv
        plsc.addupdate_scatter(acc, [idx], vsrc * wdv)
      return c
    lax.fori_loop(0, CPW, estep, 0)

  npt = NP // NS

  def nstep(i, c):
    base = sid * npt + i * L
    wdv = wd_l[pl.ds(base, L)]
    vv = vs_l[pl.ds(base, L)]
    bv = lax.bitwise_and(plsc.bitcast(wdv, jnp.int32), m127)
    idx = iota * SEG + bv
    plsc.addupdate_scatter(acc, [idx + AW], wdv * vv)
    plsc.addupdate_scatter(acc, [idx + 2 * AW], ones)
    return c
  lax.fori_loop(0, npt // L, nstep, 0)

  def mstep(t, c):
    pltpu.sync_copy(acc.at[pl.ds(t * MCH, MCH)], shared.at[ident.at[t]],
                    add=True)
    return c
  lax.fori_loop(0, ACCP // MCH, mstep, 0)
  plsc.subcore_barrier()

  @pl.when(sid == 0)
  def _():
    pltpu.sync_copy(shared, fin)
    dacc = zeros
    for c in range(D // L):
      dacc = dacc + cb_l[pl.ds(c * L, L)] * cb_l[pl.ds(D + c * L, L)]
    cbw = jnp.sum(dacc)
    bfcs = cb_l[pl.ds(2 * D, L)][0]
    for i in range(OUTP // L):
      pe = zeros
      ps = zeros
      cnt = zeros
      for s in range(L):
        pe = pe + fin[pl.ds(s * SEG + i * L, L)]
        ps = ps + fin[pl.ds(AW + s * SEG + i * L, L)]
        cnt = cnt + fin[pl.ds(2 * AW + s * SEG + i * L, L)]
      m = (pe + ps) / jnp.maximum(cnt, 1.0)
      outm[pl.ds(i * L, L)] = m + jnp.where(cnt > 0, cbw, 0.0) + bfcs
    pltpu.async_copy(outm, m_out, sem).wait()


def _fin_call(vs, wd, srcs_f, dsts_f, cbv):
  f = pl.kernel(
      _fin_body,
      out_type=jax.ShapeDtypeStruct((OUTP,), jnp.float32),
      mesh=_mesh1,
      compiler_params=pltpu.CompilerParams(needs_layout_passes=False),
      scratch_types=[
          pltpu.VMEM((NP,), jnp.float32),
          pltpu.VMEM((NP,), jnp.float32),
          pltpu.VMEM((EPW,), jnp.int32),
          pltpu.VMEM((EPW,), jnp.int32),
          pltpu.VMEM((ACCP,), jnp.float32),
          pltpu.VMEM((ACCP // MCH, MCH), jnp.int32),
          pltpu.VMEM((ACCP,), jnp.float32),
          pltpu.VMEM((CBN,), jnp.float32),
          pltpu.VMEM((OUTP,), jnp.float32),
          pltpu.SemaphoreType.DMA,
          pltpu.VMEM_SHARED((ACCP,), jnp.float32),
      ],
  )
  return f(vs, wd, srcs_f, dsts_f, cbv)


@jax.jit
def kernel(x, edge_index, batch, W1, b1, W2, b2, Wfc, bfc):
  src = edge_index[0]
  dst = edge_index[1]
  pad = jnp.full((EPAD - E,), N, dtype=jnp.int32)
  src_f = jnp.concatenate([src, pad])
  dst_f = jnp.concatenate([dst, pad])
  srcs3 = src_f.reshape(NW, CPW, CH)
  dsts3 = dst_f.reshape(NW, CPW, CH)
  xpad = jnp.pad(x, ((0, NP - N), (0, 0)))
  batchp = jnp.pad(batch, (0, NP - N), constant_values=G)
  cbv = jnp.concatenate(
      [b2, Wfc.reshape(D), jnp.broadcast_to(bfc, (CBN - 2 * D,))])

  deg_flat = _deg_call(dsts3)
  degT = deg_flat.reshape(NC, NP).T
  xs, dinv, wd = _tc1_call(degT, xpad, batchp.reshape(NP, 1))
  s_flat = _gat_call(xs, src_f.reshape(TOTC, CH), dst_f.reshape(TOTC, CH))
  vs2 = _tc2_call(s_flat, xs, dinv, W1, b1.reshape(1, D), W2, Wfc)
  m = _fin_call(vs2.reshape(NP), wd.reshape(NP), src_f, dst_f, cbv)
  return m[:G][:, None]


# speedup vs baseline: 1.1210x; 1.0337x over previous
import functools

import jax
import jax.numpy as jnp
from jax import lax
from jax.experimental import pallas as pl
from jax.experimental.pallas import tpu as pltpu
from jax.experimental.pallas import tpu_sc as plsc

N = 10000
E = 320000
D = 128
G = 64

L = 16
NC = 2
NS = 16
NW = NC * NS

NP = 10240
CH = 64
GS = 32
CPW = 160
EPW = CPW * CH
EPAD = NW * EPW
RPT = NP // NS
MCH = 128

SEG = 80
AW = L * SEG
ACCP = 3 * AW
OUTP = 80
CBN = 2 * D + L

_mesh2 = plsc.VectorSubcoreMesh(
    core_axis_name="c", subcore_axis_name="s", num_cores=NC, num_subcores=NS)
_mesh1 = plsc.VectorSubcoreMesh(
    core_axis_name="c", subcore_axis_name="s", num_cores=1, num_subcores=NS)


def _deg_body(dsts, deg_out, didx, ones_b, zb, sem, shared):
  cid = lax.axis_index("c")
  sid = lax.axis_index("s")
  w = sid * NC + cid
  zeros = jnp.zeros((L,), jnp.float32)
  ones = jnp.ones((L,), jnp.float32)
  for i in range(RPT // L):
    zb[pl.ds(i * L, L)] = zeros
  for i in range(CH // L):
    ones_b[pl.ds(i * L, L)] = ones
  pltpu.sync_copy(zb, shared.at[pl.ds(sid * RPT, RPT)])
  pltpu.sync_copy(dsts.at[w], didx)
  plsc.subcore_barrier()

  def step(j, c):
    pltpu.sync_copy(ones_b, shared.at[didx.at[j]], add=True)
    return c
  lax.fori_loop(0, CPW, step, 0)
  plsc.subcore_barrier()

  @pl.when(sid == 0)
  def _():
    pltpu.async_copy(shared, deg_out.at[pl.ds(cid * NP, NP)], sem).wait()


def _deg_call(dsts3):
  f = pl.kernel(
      _deg_body,
      out_type=jax.ShapeDtypeStruct((NC * NP,), jnp.float32),
      mesh=_mesh2,
      scratch_types=[
          pltpu.VMEM((CPW, CH), jnp.int32),
          pltpu.VMEM((CH,), jnp.float32),
          pltpu.VMEM((RPT,), jnp.float32),
          pltpu.SemaphoreType.DMA,
          pltpu.VMEM_SHARED((NP,), jnp.float32),
      ],
  )
  return f(dsts3)


def _tc1_body(degT, x, bp, xs_out, dinv_out, wd_out):
  d = degT[:, 0:1] + degT[:, 1:2] + 1.0
  dinv = lax.rsqrt(d)
  dinv_out[...] = dinv
  bits = lax.bitcast_convert_type(dinv, jnp.int32)
  wd_out[...] = lax.bitcast_convert_type(
      lax.bitwise_or(lax.bitwise_and(bits, -128), bp[...]), jnp.float32)
  xs_out[...] = x[...] * dinv


def _tc1_call(degT, xpad, batchp2):
  blk = NP // 10
  return pl.pallas_call(
      _tc1_body,
      grid=(10,),
      in_specs=[
          pl.BlockSpec((blk, 2), lambda i: (i, 0)),
          pl.BlockSpec((blk, D), lambda i: (i, 0)),
          pl.BlockSpec((blk, 1), lambda i: (i, 0)),
      ],
      out_specs=[
          pl.BlockSpec((blk, D), lambda i: (i, 0)),
          pl.BlockSpec((blk, 1), lambda i: (i, 0)),
          pl.BlockSpec((blk, 1), lambda i: (i, 0)),
      ],
      out_shape=[
          jax.ShapeDtypeStruct((NP, D), jnp.float32),
          jax.ShapeDtypeStruct((NP, 1), jnp.float32),
          jax.ShapeDtypeStruct((NP, 1), jnp.float32),
      ],
  )(degT, xpad, batchp2)


_NBUF = 4


def _gat_body(xs_h, srcs, dsts, s_out, sidx, didx, r0, r1, r2, r3, zb, sem,
              gsem, ssem, shared):
  cid = lax.axis_index("c")
  sid = lax.axis_index("s")
  w = sid * NC + cid
  zeros = jnp.zeros((L,), jnp.float32)
  for r in range(16):
    for c2 in range(D // L):
      zb[r, pl.ds(c2 * L, L)] = zeros

  def zstep(t, c):
    pltpu.sync_copy(zb, shared.at[pl.ds(sid * RPT + t * 16, 16)])
    return c
  lax.fori_loop(0, RPT // 16, zstep, 0)
  plsc.subcore_barrier()

  rows = [r0, r1, r2, r3]

  def _gth(j, b):
    return pltpu.make_async_copy(xs_h.at[sidx.at[j]], rows[b], gsem)

  def _sct(j, b):
    return pltpu.make_async_copy(rows[b], shared.at[didx.at[j]], ssem)

  def ggroup(g, c):
    pltpu.sync_copy(srcs.at[w, pl.ds(g * GS, GS)], sidx)
    pltpu.sync_copy(dsts.at[w, pl.ds(g * GS, GS)], didx)
    for b in range(_NBUF):
      _gth(b, b).start()

    def sub(sg, c2):
      base = sg * _NBUF
      for b in range(_NBUF):
        _gth(base + b, b).wait()
        _sct(base + b, b).start(add=True)
      for b in range(_NBUF):
        _sct(base + b, b).wait()
        _gth(base + _NBUF + b, b).start()
      return c2
    lax.fori_loop(0, GS // _NBUF - 1, sub, 0)
    base = GS - _NBUF
    for b in range(_NBUF):
      _gth(base + b, b).wait()
      _sct(base + b, b).start(add=True)
    for b in range(_NBUF):
      _sct(base + b, b).wait()
    return c
  lax.fori_loop(0, CPW // GS, ggroup, 0)
  plsc.subcore_barrier()
  pltpu.async_copy(
      shared.at[pl.ds(sid * RPT, RPT)],
      s_out.at[pl.ds(cid * NP + sid * RPT, RPT)], sem).wait()


def _gat_call(xs, srcs3, dsts3):
  f = pl.kernel(
      _gat_body,
      out_type=jax.ShapeDtypeStruct((NC * NP, D), jnp.float32),
      mesh=_mesh2,
      scratch_types=[
          pltpu.VMEM((GS, CH), jnp.int32),
          pltpu.VMEM((GS, CH), jnp.int32),
          pltpu.VMEM((CH, D), jnp.float32),
          pltpu.VMEM((CH, D), jnp.float32),
          pltpu.VMEM((CH, D), jnp.float32),
          pltpu.VMEM((CH, D), jnp.float32),
          pltpu.VMEM((16, D), jnp.float32),
          pltpu.SemaphoreType.DMA,
          pltpu.SemaphoreType.DMA,
          pltpu.SemaphoreType.DMA,
          pltpu.VMEM_SHARED((NP, D), jnp.float32),
      ],
  )
  return f(xs, srcs3, dsts3)


def _tc2_body(s0, s1, xs, dinvc, w1, b1r, w2, wfc, out):
  hi = jax.lax.Precision.HIGHEST
  u = jnp.dot(w2[...], wfc[...], precision=hi,
              preferred_element_type=jnp.float32)
  a = (s0[...] + s1[...] + xs[...]) * dinvc[...]
  h1 = jnp.dot(a, w1[...], precision=hi,
               preferred_element_type=jnp.float32) + b1r[...]
  h1 = jnp.maximum(h1, 0.0)
  v = jnp.dot(h1, u, precision=hi, preferred_element_type=jnp.float32)
  out[...] = v * dinvc[...]


def _tc2_call(s_flat, xs, dinv, W1, b1r, W2, Wfc):
  blk = NP // 10
  nb = NP // blk
  return pl.pallas_call(
      _tc2_body,
      grid=(10,),
      in_specs=[
          pl.BlockSpec((blk, D), lambda i: (i, 0)),
          pl.BlockSpec((blk, D), lambda i, _nb=nb: (i + _nb, 0)),
          pl.BlockSpec((blk, D), lambda i: (i, 0)),
          pl.BlockSpec((blk, 1), lambda i: (i, 0)),
          pl.BlockSpec((D, D), lambda i: (0, 0)),
          pl.BlockSpec((1, D), lambda i: (0, 0)),
          pl.BlockSpec((D, D), lambda i: (0, 0)),
          pl.BlockSpec((D, 1), lambda i: (0, 0)),
      ],
      out_specs=pl.BlockSpec((blk, 1), lambda i: (i, 0)),
      out_shape=jax.ShapeDtypeStruct((NP, 1), jnp.float32),
  )(s_flat, s_flat, xs, dinv, W1, b1r, W2, Wfc)


def _fin_body(vs_h, wd_h, srcs_f, dsts_f, cb_h, m_out, vs_l, wd_l, sidx,
              didx, acc, ident, fin, cb_l, outm, sem, shared):
  sid = lax.axis_index("s")
  zeros = jnp.zeros((L,), jnp.float32)
  ones = jnp.ones((L,), jnp.float32)
  iota = lax.iota(jnp.int32, L)
  m127 = jnp.full((L,), 127, jnp.int32)

  def zstep(i, c):
    acc[pl.ds(i * L, L)] = zeros
    return c
  lax.fori_loop(0, ACCP // L, zstep, 0)
  stripe = ACCP // NS
  pltpu.sync_copy(acc.at[pl.ds(0, stripe)],
                  shared.at[pl.ds(sid * stripe, stripe)])
  for t in range(ACCP // MCH):
    for kk in range(MCH // L):
      ident[t, pl.ds(kk * L, L)] = t * MCH + kk * L + iota
  pltpu.sync_copy(vs_h, vs_l)
  pltpu.sync_copy(wd_h, wd_l)
  pltpu.sync_copy(cb_h, cb_l)
  plsc.subcore_barrier()

  for half in range(2):
    w = sid * 2 + half
    pltpu.sync_copy(srcs_f.at[pl.ds(w * EPW, EPW)], sidx)
    pltpu.sync_copy(dsts_f.at[pl.ds(w * EPW, EPW)], didx)

    def estep(j, c):
      for kk in range(CH // L):
        off = j * CH + kk * L
        sv = sidx[pl.ds(off, L)]
        dv = didx[pl.ds(off, L)]
        vsrc = plsc.load_gather(vs_l, [sv])
        wdv = plsc.load_gather(wd_l, [dv])
        bv = lax.bitwise_and(plsc.bitcast(wdv, jnp.int32), m127)
        idx = iota * SEG + bv
        plsc.addupdate_scatter(acc, [idx], vsrc * wdv)
      return c
    lax.fori_loop(0, CPW, estep, 0)

  npt = NP // NS

  def nstep(i, c):
    base = sid * npt + i * L
    wdv = wd_l[pl.ds(base, L)]
    vv = vs_l[pl.ds(base, L)]
    bv = lax.bitwise_and(plsc.bitcast(wdv, jnp.int32), m127)
    idx = iota * SEG + bv
    plsc.addupdate_scatter(acc, [idx + AW], wdv * vv)
    plsc.addupdate_scatter(acc, [idx + 2 * AW], ones)
    return c
  lax.fori_loop(0, npt // L, nstep, 0)

  def mstep(t, c):
    pltpu.sync_copy(acc.at[pl.ds(t * MCH, MCH)], shared.at[ident.at[t]],
                    add=True)
    return c
  lax.fori_loop(0, ACCP // MCH, mstep, 0)
  plsc.subcore_barrier()

  @pl.when(sid == 0)
  def _():
    pltpu.sync_copy(shared, fin)
    dacc = zeros
    for c in range(D // L):
      dacc = dacc + cb_l[pl.ds(c * L, L)] * cb_l[pl.ds(D + c * L, L)]
    cbw = jnp.sum(dacc)
    bfcs = cb_l[pl.ds(2 * D, L)][0]
    for i in range(OUTP // L):
      pe = zeros
      ps = zeros
      cnt = zeros
      for s in range(L):
        pe = pe + fin[pl.ds(s * SEG + i * L, L)]
        ps = ps + fin[pl.ds(AW + s * SEG + i * L, L)]
        cnt = cnt + fin[pl.ds(2 * AW + s * SEG + i * L, L)]
      m = (pe + ps) / jnp.maximum(cnt, 1.0)
      outm[pl.ds(i * L, L)] = m + jnp.where(cnt > 0, cbw, 0.0) + bfcs
    pltpu.async_copy(outm, m_out, sem).wait()


def _fin_call(vs, wd, srcs_f, dsts_f, cbv):
  f = pl.kernel(
      _fin_body,
      out_type=jax.ShapeDtypeStruct((OUTP,), jnp.float32),
      mesh=_mesh1,
      compiler_params=pltpu.CompilerParams(needs_layout_passes=False),
      scratch_types=[
          pltpu.VMEM((NP,), jnp.float32),
          pltpu.VMEM((NP,), jnp.float32),
          pltpu.VMEM((EPW,), jnp.int32),
          pltpu.VMEM((EPW,), jnp.int32),
          pltpu.VMEM((ACCP,), jnp.float32),
          pltpu.VMEM((ACCP // MCH, MCH), jnp.int32),
          pltpu.VMEM((ACCP,), jnp.float32),
          pltpu.VMEM((CBN,), jnp.float32),
          pltpu.VMEM((OUTP,), jnp.float32),
          pltpu.SemaphoreType.DMA,
          pltpu.VMEM_SHARED((ACCP,), jnp.float32),
      ],
  )
  return f(vs, wd, srcs_f, dsts_f, cbv)


@jax.jit
def kernel(x, edge_index, batch, W1, b1, W2, b2, Wfc, bfc):
  src = edge_index[0]
  dst = edge_index[1]
  pad = jnp.full((EPAD - E,), N, dtype=jnp.int32)
  src_f = jnp.concatenate([src, pad])
  dst_f = jnp.concatenate([dst, pad])
  srcs3 = src_f.reshape(NW, CPW, CH)
  dsts3 = dst_f.reshape(NW, CPW, CH)
  xpad = jnp.pad(x, ((0, NP - N), (0, 0)))
  batchp = jnp.pad(batch, (0, NP - N), constant_values=G)
  cbv = jnp.concatenate(
      [b2, Wfc.reshape(D), jnp.broadcast_to(bfc, (CBN - 2 * D,))])

  deg_flat = _deg_call(dsts3)
  degT = deg_flat.reshape(NC, NP).T
  xs, dinv, wd = _tc1_call(degT, xpad, batchp.reshape(NP, 1))
  s_flat = _gat_call(xs, srcs3, dsts3)
  vs2 = _tc2_call(s_flat, xs, dinv, W1, b1.reshape(1, D), W2, Wfc)
  m = _fin_call(vs2.reshape(NP), wd.reshape(NP), src_f, dst_f, cbv)
  return m[:G][:, None]


# speedup vs baseline: 1.2222x; 1.0903x over previous
import functools

import jax
import jax.numpy as jnp
from jax import lax
from jax.experimental import pallas as pl
from jax.experimental.pallas import tpu as pltpu
from jax.experimental.pallas import tpu_sc as plsc

N = 10000
E = 320000
D = 128
G = 64

L = 16
NC = 2
NS = 16
NW = NC * NS

NP = 10240
CH = 64
GS = 32
CPW = 160
EPW = CPW * CH
EPAD = NW * EPW
RPT = NP // NS
MCH = 128

SEG = 80
AW = L * SEG
ACCP = 3 * AW
OUTP = 80

_mesh2 = plsc.VectorSubcoreMesh(
    core_axis_name="c", subcore_axis_name="s", num_cores=NC, num_subcores=NS)
_mesh1 = plsc.VectorSubcoreMesh(
    core_axis_name="c", subcore_axis_name="s", num_cores=1, num_subcores=NS)


def _deg_body(dsts, deg_out, didx, ones_b, zb, sem, shared):
  cid = lax.axis_index("c")
  sid = lax.axis_index("s")
  w = sid * NC + cid
  zeros = jnp.zeros((L,), jnp.float32)
  ones = jnp.ones((L,), jnp.float32)
  for i in range(RPT // L):
    zb[pl.ds(i * L, L)] = zeros
  for i in range(CH // L):
    ones_b[pl.ds(i * L, L)] = ones
  pltpu.sync_copy(zb, shared.at[pl.ds(sid * RPT, RPT)])
  pltpu.sync_copy(dsts.at[w], didx)
  plsc.subcore_barrier()

  def step(j, c):
    pltpu.sync_copy(ones_b, shared.at[didx.at[j]], add=True)
    return c
  lax.fori_loop(0, CPW, step, 0)
  plsc.subcore_barrier()

  @pl.when(sid == 0)
  def _():
    pltpu.async_copy(shared, deg_out.at[pl.ds(cid * NP, NP)], sem).wait()


def _deg_call(dsts3):
  f = pl.kernel(
      _deg_body,
      out_type=jax.ShapeDtypeStruct((NC * NP,), jnp.float32),
      mesh=_mesh2,
      scratch_types=[
          pltpu.VMEM((CPW, CH), jnp.int32),
          pltpu.VMEM((CH,), jnp.float32),
          pltpu.VMEM((RPT,), jnp.float32),
          pltpu.SemaphoreType.DMA,
          pltpu.VMEM_SHARED((NP,), jnp.float32),
      ],
  )
  return f(dsts3)


def _tc1_body(degT, x, xs_out, dinv_out):
  d = degT[:, 0:1] + degT[:, 1:2] + 1.0
  dinv = lax.rsqrt(d)
  dinv_out[...] = dinv
  xs_out[...] = x[...] * dinv


def _tc1_call(degT, xpad):
  blk = NP // 10
  return pl.pallas_call(
      _tc1_body,
      grid=(10,),
      in_specs=[
          pl.BlockSpec((blk, 2), lambda i: (i, 0)),
          pl.BlockSpec((blk, D), lambda i: (i, 0)),
      ],
      out_specs=[
          pl.BlockSpec((blk, D), lambda i: (i, 0)),
          pl.BlockSpec((blk, 1), lambda i: (i, 0)),
      ],
      out_shape=[
          jax.ShapeDtypeStruct((NP, D), jnp.float32),
          jax.ShapeDtypeStruct((NP, 1), jnp.float32),
      ],
  )(degT, xpad)


_NBUF = 4


def _gat_body(xs_h, srcs, dsts, s_out, sidx, didx, r0, r1, r2, r3, zb, sem,
              gsem, ssem, shared):
  cid = lax.axis_index("c")
  sid = lax.axis_index("s")
  w = sid * NC + cid
  zeros = jnp.zeros((L,), jnp.float32)
  for r in range(16):
    for c2 in range(D // L):
      zb[r, pl.ds(c2 * L, L)] = zeros

  def zstep(t, c):
    pltpu.sync_copy(zb, shared.at[pl.ds(sid * RPT + t * 16, 16)])
    return c
  lax.fori_loop(0, RPT // 16, zstep, 0)
  plsc.subcore_barrier()

  rows = [r0, r1, r2, r3]

  def _gth(j, b):
    return pltpu.make_async_copy(xs_h.at[sidx.at[j]], rows[b], gsem)

  def _sct(j, b):
    return pltpu.make_async_copy(rows[b], shared.at[didx.at[j]], ssem)

  def ggroup(g, c):
    pltpu.sync_copy(srcs.at[w, pl.ds(g * GS, GS)], sidx)
    pltpu.sync_copy(dsts.at[w, pl.ds(g * GS, GS)], didx)
    for b in range(_NBUF):
      _gth(b, b).start()

    def sub(sg, c2):
      base = sg * _NBUF
      for b in range(_NBUF):
        _gth(base + b, b).wait()
        _sct(base + b, b).start(add=True)
      for b in range(_NBUF):
        _sct(base + b, b).wait()
        _gth(base + _NBUF + b, b).start()
      return c2
    lax.fori_loop(0, GS // _NBUF - 1, sub, 0)
    base = GS - _NBUF
    for b in range(_NBUF):
      _gth(base + b, b).wait()
      _sct(base + b, b).start(add=True)
    for b in range(_NBUF):
      _sct(base + b, b).wait()
    return c
  lax.fori_loop(0, CPW // GS, ggroup, 0)
  plsc.subcore_barrier()
  pltpu.async_copy(
      shared.at[pl.ds(sid * RPT, RPT)],
      s_out.at[pl.ds(cid * NP + sid * RPT, RPT)], sem).wait()


def _gat_call(xs, srcs3, dsts3):
  f = pl.kernel(
      _gat_body,
      out_type=jax.ShapeDtypeStruct((NC * NP, D), jnp.float32),
      mesh=_mesh2,
      scratch_types=[
          pltpu.VMEM((GS, CH), jnp.int32),
          pltpu.VMEM((GS, CH), jnp.int32),
          pltpu.VMEM((CH, D), jnp.float32),
          pltpu.VMEM((CH, D), jnp.float32),
          pltpu.VMEM((CH, D), jnp.float32),
          pltpu.VMEM((CH, D), jnp.float32),
          pltpu.VMEM((16, D), jnp.float32),
          pltpu.SemaphoreType.DMA,
          pltpu.SemaphoreType.DMA,
          pltpu.SemaphoreType.DMA,
          pltpu.VMEM_SHARED((NP, D), jnp.float32),
      ],
  )
  return f(xs, srcs3, dsts3)


def _tc2_body(s0, s1, xs, dinvc, w1, b1r, w2, wfc, out):
  hi = jax.lax.Precision.HIGHEST
  u = jnp.dot(w2[...], wfc[...], precision=hi,
              preferred_element_type=jnp.float32)
  a = (s0[...] + s1[...] + xs[...]) * dinvc[...]
  h1 = jnp.dot(a, w1[...], precision=hi,
               preferred_element_type=jnp.float32) + b1r[...]
  h1 = jnp.maximum(h1, 0.0)
  v = jnp.dot(h1, u, precision=hi, preferred_element_type=jnp.float32)
  out[...] = v * dinvc[...]


def _tc2_call(s_flat, xs, dinv, W1, b1r, W2, Wfc):
  blk = NP // 10
  nb = NP // blk
  return pl.pallas_call(
      _tc2_body,
      grid=(10,),
      in_specs=[
          pl.BlockSpec((blk, D), lambda i: (i, 0)),
          pl.BlockSpec((blk, D), lambda i, _nb=nb: (i + _nb, 0)),
          pl.BlockSpec((blk, D), lambda i: (i, 0)),
          pl.BlockSpec((blk, 1), lambda i: (i, 0)),
          pl.BlockSpec((D, D), lambda i: (0, 0)),
          pl.BlockSpec((1, D), lambda i: (0, 0)),
          pl.BlockSpec((D, D), lambda i: (0, 0)),
          pl.BlockSpec((D, 1), lambda i: (0, 0)),
      ],
      out_specs=pl.BlockSpec((blk, 1), lambda i: (i, 0)),
      out_shape=jax.ShapeDtypeStruct((NP, 1), jnp.float32),
  )(s_flat, s_flat, xs, dinv, W1, b1r, W2, Wfc)


def _fin_body(vs_h, dinv_h, batch_h, srcs_f, dsts_f, m_out, c_out, vs_l,
              dinv_l, batch_l, sidx, didx, acc, ident, fin, outm, outc, sem,
              shared):
  sid = lax.axis_index("s")
  zeros = jnp.zeros((L,), jnp.float32)
  ones = jnp.ones((L,), jnp.float32)
  iota = lax.iota(jnp.int32, L)

  def zstep(i, c):
    acc[pl.ds(i * L, L)] = zeros
    return c
  lax.fori_loop(0, ACCP // L, zstep, 0)
  stripe = ACCP // NS
  pltpu.sync_copy(acc.at[pl.ds(0, stripe)],
                  shared.at[pl.ds(sid * stripe, stripe)])
  for t in range(ACCP // MCH):
    for kk in range(MCH // L):
      ident[t, pl.ds(kk * L, L)] = t * MCH + kk * L + iota
  pltpu.sync_copy(vs_h, vs_l)
  pltpu.sync_copy(dinv_h, dinv_l)
  pltpu.sync_copy(batch_h, batch_l)
  plsc.subcore_barrier()

  for half in range(2):
    w = sid * 2 + half
    pltpu.sync_copy(srcs_f.at[pl.ds(w * EPW, EPW)], sidx)
    pltpu.sync_copy(dsts_f.at[pl.ds(w * EPW, EPW)], didx)

    def estep(j, c):
      for kk in range(CH // L):
        off = j * CH + kk * L
        sv = sidx[pl.ds(off, L)]
        dv = didx[pl.ds(off, L)]
        vsrc = plsc.load_gather(vs_l, [sv])
        ddv = plsc.load_gather(dinv_l, [dv])
        bv = plsc.load_gather(batch_l, [dv])
        idx = iota * SEG + bv
        plsc.addupdate_scatter(acc, [idx], vsrc * ddv)
      return c
    lax.fori_loop(0, CPW, estep, 0)

  npt = NP // NS

  def nstep(i, c):
    base = sid * npt + i * L
    dv = dinv_l[pl.ds(base, L)]
    vv = vs_l[pl.ds(base, L)]
    bv = batch_l[pl.ds(base, L)]
    idx = iota * SEG + bv
    plsc.addupdate_scatter(acc, [idx + AW], dv * vv)
    plsc.addupdate_scatter(acc, [idx + 2 * AW], ones)
    return c
  lax.fori_loop(0, npt // L, nstep, 0)

  def mstep(t, c):
    pltpu.sync_copy(acc.at[pl.ds(t * MCH, MCH)], shared.at[ident.at[t]],
                    add=True)
    return c
  lax.fori_loop(0, ACCP // MCH, mstep, 0)
  plsc.subcore_barrier()

  @pl.when(sid == 0)
  def _():
    pltpu.sync_copy(shared, fin)
    for i in range(OUTP // L):
      pe = zeros
      ps = zeros
      cnt = zeros
      for s in range(L):
        pe = pe + fin[pl.ds(s * SEG + i * L, L)]
        ps = ps + fin[pl.ds(AW + s * SEG + i * L, L)]
        cnt = cnt + fin[pl.ds(2 * AW + s * SEG + i * L, L)]
      outm[pl.ds(i * L, L)] = (pe + ps) / jnp.maximum(cnt, 1.0)
      outc[pl.ds(i * L, L)] = cnt
    pltpu.async_copy(outm, m_out, sem).wait()
    pltpu.async_copy(outc, c_out, sem).wait()


def _fin_call(vs, dinv, batchp, srcs_f, dsts_f):
  f = pl.kernel(
      _fin_body,
      out_type=[
          jax.ShapeDtypeStruct((OUTP,), jnp.float32),
          jax.ShapeDtypeStruct((OUTP,), jnp.float32),
      ],
      mesh=_mesh1,
      compiler_params=pltpu.CompilerParams(needs_layout_passes=False),
      scratch_types=[
          pltpu.VMEM((NP,), jnp.float32),
          pltpu.VMEM((NP,), jnp.float32),
          pltpu.VMEM((NP,), jnp.int32),
          pltpu.VMEM((EPW,), jnp.int32),
          pltpu.VMEM((EPW,), jnp.int32),
          pltpu.VMEM((ACCP,), jnp.float32),
          pltpu.VMEM((ACCP // MCH, MCH), jnp.int32),
          pltpu.VMEM((ACCP,), jnp.float32),
          pltpu.VMEM((OUTP,), jnp.float32),
          pltpu.VMEM((OUTP,), jnp.float32),
          pltpu.SemaphoreType.DMA,
          pltpu.VMEM_SHARED((ACCP,), jnp.float32),
      ],
  )
  return f(vs, dinv, batchp, srcs_f, dsts_f)


@jax.jit
def kernel(x, edge_index, batch, W1, b1, W2, b2, Wfc, bfc):
  src = edge_index[0]
  dst = edge_index[1]
  pad = jnp.full((EPAD - E,), N, dtype=jnp.int32)
  src_f = jnp.concatenate([src, pad])
  dst_f = jnp.concatenate([dst, pad])
  srcs3 = src_f.reshape(NW, CPW, CH)
  dsts3 = dst_f.reshape(NW, CPW, CH)
  xpad = jnp.pad(x, ((0, NP - N), (0, 0)))
  batchp = jnp.pad(batch, (0, NP - N), constant_values=G)

  deg_flat = _deg_call(dsts3)
  degT = deg_flat.reshape(NC, NP).T
  xs, dinv = _tc1_call(degT, xpad)
  s_flat = _gat_call(xs, srcs3, dsts3)
  vs2 = _tc2_call(s_flat, xs, dinv, W1, b1.reshape(1, D), W2, Wfc)
  m, cnt = _fin_call(vs2.reshape(NP), dinv.reshape(NP), batchp, src_f, dst_f)

  cb = (b2 @ Wfc)[0]
  out = m[:G] + jnp.where(cnt[:G] > 0, cb, 0.0) + bfc[0]
  return out[:, None]


# speedup vs baseline: 1.2323x; 1.0083x over previous
import functools

import jax
import jax.numpy as jnp
from jax import lax
from jax.experimental import pallas as pl
from jax.experimental.pallas import tpu as pltpu
from jax.experimental.pallas import tpu_sc as plsc

N = 10000
E = 320000
D = 128
G = 64

L = 16
NC = 2
NS = 16
NW = NC * NS

NP = 10240
CH = 64
GS = 32
CPW = 160
EPW = CPW * CH
EPAD = NW * EPW
RPT = NP // NS
CHA = 128
CPWA = EPW // CHA
MCH = 128

SEG = 80
AW = L * SEG
ACCP = 3 * AW
OUTP = 80

_mesh2 = plsc.VectorSubcoreMesh(
    core_axis_name="c", subcore_axis_name="s", num_cores=NC, num_subcores=NS)
_mesh1 = plsc.VectorSubcoreMesh(
    core_axis_name="c", subcore_axis_name="s", num_cores=1, num_subcores=NS)


def _deg_body(dsts, deg_out, didx, ones_b, zb, sem, shared):
  cid = lax.axis_index("c")
  sid = lax.axis_index("s")
  w = sid * NC + cid
  zeros = jnp.zeros((L,), jnp.float32)
  ones = jnp.ones((L,), jnp.float32)
  for i in range(RPT // L):
    zb[pl.ds(i * L, L)] = zeros
  for i in range(CHA // L):
    ones_b[pl.ds(i * L, L)] = ones
  pltpu.sync_copy(zb, shared.at[pl.ds(sid * RPT, RPT)])
  pltpu.sync_copy(dsts.at[w], didx)
  plsc.subcore_barrier()

  def step(j, c):
    pltpu.sync_copy(ones_b, shared.at[didx.at[j]], add=True)
    return c
  lax.fori_loop(0, CPWA, step, 0)
  plsc.subcore_barrier()

  @pl.when(sid == 0)
  def _():
    pltpu.async_copy(shared, deg_out.at[pl.ds(cid * NP, NP)], sem).wait()


def _deg_call(dsts3):
  f = pl.kernel(
      _deg_body,
      out_type=jax.ShapeDtypeStruct((NC * NP,), jnp.float32),
      mesh=_mesh2,
      scratch_types=[
          pltpu.VMEM((CPWA, CHA), jnp.int32),
          pltpu.VMEM((CHA,), jnp.float32),
          pltpu.VMEM((RPT,), jnp.float32),
          pltpu.SemaphoreType.DMA,
          pltpu.VMEM_SHARED((NP,), jnp.float32),
      ],
  )
  return f(dsts3)


def _tc1_body(degT, x, xs_out, dinv_out):
  d = degT[:, 0:1] + degT[:, 1:2] + 1.0
  dinv = lax.rsqrt(d)
  dinv_out[...] = dinv
  xs_out[...] = x[...] * dinv


def _tc1_call(degT, xpad):
  blk = NP // 10
  return pl.pallas_call(
      _tc1_body,
      grid=(10,),
      in_specs=[
          pl.BlockSpec((blk, 2), lambda i: (i, 0)),
          pl.BlockSpec((blk, D), lambda i: (i, 0)),
      ],
      out_specs=[
          pl.BlockSpec((blk, D), lambda i: (i, 0)),
          pl.BlockSpec((blk, 1), lambda i: (i, 0)),
      ],
      out_shape=[
          jax.ShapeDtypeStruct((NP, D), jnp.float32),
          jax.ShapeDtypeStruct((NP, 1), jnp.float32),
      ],
  )(degT, xpad)


_NBUF = 4


def _gat_body(xs_h, srcs, dsts, s_out, sidx, didx, r0, r1, r2, r3, zb, sem,
              gsem, ssem, shared):
  cid = lax.axis_index("c")
  sid = lax.axis_index("s")
  w = sid * NC + cid
  zeros = jnp.zeros((L,), jnp.float32)
  for r in range(16):
    for c2 in range(D // L):
      zb[r, pl.ds(c2 * L, L)] = zeros

  def zstep(t, c):
    pltpu.sync_copy(zb, shared.at[pl.ds(sid * RPT + t * 16, 16)])
    return c
  lax.fori_loop(0, RPT // 16, zstep, 0)
  plsc.subcore_barrier()

  rows = [r0, r1, r2, r3]

  def _gth(j, b):
    return pltpu.make_async_copy(xs_h.at[sidx.at[j]], rows[b], gsem)

  def _sct(j, b):
    return pltpu.make_async_copy(rows[b], shared.at[didx.at[j]], ssem)

  def ggroup(g, c):
    pltpu.sync_copy(srcs.at[w, pl.ds(g * GS, GS)], sidx)
    pltpu.sync_copy(dsts.at[w, pl.ds(g * GS, GS)], didx)
    for b in range(_NBUF):
      _gth(b, b).start()

    def sub(sg, c2):
      base = sg * _NBUF
      for b in range(_NBUF):
        _gth(base + b, b).wait()
        _sct(base + b, b).start(add=True)
      for b in range(_NBUF):
        _sct(base + b, b).wait()
        _gth(base + _NBUF + b, b).start()
      return c2
    lax.fori_loop(0, GS // _NBUF - 1, sub, 0)
    base = GS - _NBUF
    for b in range(_NBUF):
      _gth(base + b, b).wait()
      _sct(base + b, b).start(add=True)
    for b in range(_NBUF):
      _sct(base + b, b).wait()
    return c
  lax.fori_loop(0, CPW // GS, ggroup, 0)
  plsc.subcore_barrier()
  pltpu.async_copy(
      shared.at[pl.ds(sid * RPT, RPT)],
      s_out.at[pl.ds(cid * NP + sid * RPT, RPT)], sem).wait()


def _gat_call(xs, srcs3, dsts3):
  f = pl.kernel(
      _gat_body,
      out_type=jax.ShapeDtypeStruct((NC * NP, D), jnp.float32),
      mesh=_mesh2,
      scratch_types=[
          pltpu.VMEM((GS, CH), jnp.int32),
          pltpu.VMEM((GS, CH), jnp.int32),
          pltpu.VMEM((CH, D), jnp.float32),
          pltpu.VMEM((CH, D), jnp.float32),
          pltpu.VMEM((CH, D), jnp.float32),
          pltpu.VMEM((CH, D), jnp.float32),
          pltpu.VMEM((16, D), jnp.float32),
          pltpu.SemaphoreType.DMA,
          pltpu.SemaphoreType.DMA,
          pltpu.SemaphoreType.DMA,
          pltpu.VMEM_SHARED((NP, D), jnp.float32),
      ],
  )
  return f(xs, srcs3, dsts3)


def _tc2_body(s0, s1, xs, dinvc, w1, b1r, w2, wfc, out):
  hi = jax.lax.Precision.HIGHEST
  u = jnp.dot(w2[...], wfc[...], precision=hi,
              preferred_element_type=jnp.float32)
  a = (s0[...] + s1[...] + xs[...]) * dinvc[...]
  h1 = jnp.dot(a, w1[...], precision=hi,
               preferred_element_type=jnp.float32) + b1r[...]
  h1 = jnp.maximum(h1, 0.0)
  v = jnp.dot(h1, u, precision=hi, preferred_element_type=jnp.float32)
  out[...] = v * dinvc[...]


def _tc2_call(s_flat, xs, dinv, W1, b1r, W2, Wfc):
  blk = NP // 10
  nb = NP // blk
  return pl.pallas_call(
      _tc2_body,
      grid=(10,),
      in_specs=[
          pl.BlockSpec((blk, D), lambda i: (i, 0)),
          pl.BlockSpec((blk, D), lambda i, _nb=nb: (i + _nb, 0)),
          pl.BlockSpec((blk, D), lambda i: (i, 0)),
          pl.BlockSpec((blk, 1), lambda i: (i, 0)),
          pl.BlockSpec((D, D), lambda i: (0, 0)),
          pl.BlockSpec((1, D), lambda i: (0, 0)),
          pl.BlockSpec((D, D), lambda i: (0, 0)),
          pl.BlockSpec((D, 1), lambda i: (0, 0)),
      ],
      out_specs=pl.BlockSpec((blk, 1), lambda i: (i, 0)),
      out_shape=jax.ShapeDtypeStruct((NP, 1), jnp.float32),
  )(s_flat, s_flat, xs, dinv, W1, b1r, W2, Wfc)


def _fin_body(vs_h, dinv_h, batch_h, srcs_f, dsts_f, m_out, c_out, vs_l,
              dinv_l, batch_l, sidx, didx, acc, ident, fin, outm, outc, sem,
              shared):
  sid = lax.axis_index("s")
  zeros = jnp.zeros((L,), jnp.float32)
  ones = jnp.ones((L,), jnp.float32)
  iota = lax.iota(jnp.int32, L)

  def zstep(i, c):
    acc[pl.ds(i * L, L)] = zeros
    return c
  lax.fori_loop(0, ACCP // L, zstep, 0)
  stripe = ACCP // NS
  pltpu.sync_copy(acc.at[pl.ds(0, stripe)],
                  shared.at[pl.ds(sid * stripe, stripe)])
  for t in range(ACCP // MCH):
    for kk in range(MCH // L):
      ident[t, pl.ds(kk * L, L)] = t * MCH + kk * L + iota
  pltpu.sync_copy(vs_h, vs_l)
  pltpu.sync_copy(dinv_h, dinv_l)
  pltpu.sync_copy(batch_h, batch_l)
  plsc.subcore_barrier()

  for half in range(2):
    w = sid * 2 + half
    pltpu.sync_copy(srcs_f.at[pl.ds(w * EPW, EPW)], sidx)
    pltpu.sync_copy(dsts_f.at[pl.ds(w * EPW, EPW)], didx)

    def estep(j, c):
      for kk in range(CH // L):
        off = j * CH + kk * L
        sv = sidx[pl.ds(off, L)]
        dv = didx[pl.ds(off, L)]
        vsrc = plsc.load_gather(vs_l, [sv])
        ddv = plsc.load_gather(dinv_l, [dv])
        bv = plsc.load_gather(batch_l, [dv])
        idx = iota * SEG + bv
        plsc.addupdate_scatter(acc, [idx], vsrc * ddv)
      return c
    lax.fori_loop(0, CPW, estep, 0)

  npt = NP // NS

  def nstep(i, c):
    base = sid * npt + i * L
    dv = dinv_l[pl.ds(base, L)]
    vv = vs_l[pl.ds(base, L)]
    bv = batch_l[pl.ds(base, L)]
    idx = iota * SEG + bv
    plsc.addupdate_scatter(acc, [idx + AW], dv * vv)
    plsc.addupdate_scatter(acc, [idx + 2 * AW], ones)
    return c
  lax.fori_loop(0, npt // L, nstep, 0)

  def mstep(t, c):
    pltpu.sync_copy(acc.at[pl.ds(t * MCH, MCH)], shared.at[ident.at[t]],
                    add=True)
    return c
  lax.fori_loop(0, ACCP // MCH, mstep, 0)
  plsc.subcore_barrier()

  @pl.when(sid == 0)
  def _():
    pltpu.sync_copy(shared, fin)
    for i in range(OUTP // L):
      pe = zeros
      ps = zeros
      cnt = zeros
      for s in range(L):
        pe = pe + fin[pl.ds(s * SEG + i * L, L)]
        ps = ps + fin[pl.ds(AW + s * SEG + i * L, L)]
        cnt = cnt + fin[pl.ds(2 * AW + s * SEG + i * L, L)]
      outm[pl.ds(i * L, L)] = (pe + ps) / jnp.maximum(cnt, 1.0)
      outc[pl.ds(i * L, L)] = cnt
    pltpu.async_copy(outm, m_out, sem).wait()
    pltpu.async_copy(outc, c_out, sem).wait()


def _fin_call(vs, dinv, batchp, srcs_f, dsts_f):
  f = pl.kernel(
      _fin_body,
      out_type=[
          jax.ShapeDtypeStruct((OUTP,), jnp.float32),
          jax.ShapeDtypeStruct((OUTP,), jnp.float32),
      ],
      mesh=_mesh1,
      compiler_params=pltpu.CompilerParams(needs_layout_passes=False),
      scratch_types=[
          pltpu.VMEM((NP,), jnp.float32),
          pltpu.VMEM((NP,), jnp.float32),
          pltpu.VMEM((NP,), jnp.int32),
          pltpu.VMEM((EPW,), jnp.int32),
          pltpu.VMEM((EPW,), jnp.int32),
          pltpu.VMEM((ACCP,), jnp.float32),
          pltpu.VMEM((ACCP // MCH, MCH), jnp.int32),
          pltpu.VMEM((ACCP,), jnp.float32),
          pltpu.VMEM((OUTP,), jnp.float32),
          pltpu.VMEM((OUTP,), jnp.float32),
          pltpu.SemaphoreType.DMA,
          pltpu.VMEM_SHARED((ACCP,), jnp.float32),
      ],
  )
  return f(vs, dinv, batchp, srcs_f, dsts_f)


@jax.jit
def kernel(x, edge_index, batch, W1, b1, W2, b2, Wfc, bfc):
  src = edge_index[0]
  dst = edge_index[1]
  pad = jnp.full((EPAD - E,), N, dtype=jnp.int32)
  src_f = jnp.concatenate([src, pad])
  dst_f = jnp.concatenate([dst, pad])
  srcs3 = src_f.reshape(NW, CPW, CH)
  dsts3 = dst_f.reshape(NW, CPW, CH)
  xpad = jnp.pad(x, ((0, NP - N), (0, 0)))
  batchp = jnp.pad(batch, (0, NP - N), constant_values=G)

  deg_flat = _deg_call(dst_f.reshape(NW, CPWA, CHA))
  degT = deg_flat.reshape(NC, NP).T
  xs, dinv = _tc1_call(degT, xpad)
  s_flat = _gat_call(xs, srcs3, dsts3)
  vs2 = _tc2_call(s_flat, xs, dinv, W1, b1.reshape(1, D), W2, Wfc)
  m, cnt = _fin_call(vs2.reshape(NP), dinv.reshape(NP), batchp, src_f, dst_f)

  cb = (b2 @ Wfc)[0]
  out = m[:G] + jnp.where(cnt[:G] > 0, cb, 0.0) + bfc[0]
  return out[:, None]
